# baseline (speedup 1.0000x reference)
import functools

import jax
import jax.numpy as jnp
from jax import lax
from jax.experimental import pallas as pl
from jax.experimental.pallas import tpu as pltpu
from jax.experimental.pallas import tpu_sc as plsc

_L = 16
_CH = 128
_NB = 4


@functools.cache
def _build(NI, NJ, V, D):
    info = plsc.get_sparse_core_info()
    NC, NS = info.num_cores, info.num_subcores
    NW = NC * NS
    assert NI == NW * _CH and NJ % (2 * _NB) == 0 and D == 64
    mesh = plsc.VectorSubcoreMesh(core_axis_name="c", subcore_axis_name="s")

    @functools.partial(
        pl.kernel,
        out_type=jax.ShapeDtypeStruct((NJ, D // 8, NW, 8, _CH), jnp.float32),
        mesh=mesh,
        scratch_types=[
            pltpu.VMEM((NJ // 8, 8, _CH), jnp.int32),
            pltpu.VMEM((_NB, _CH), jnp.int32),
            pltpu.VMEM((_NB, _CH, D), jnp.float32),
            pltpu.VMEM((_NB, D // 8, 8, _CH), jnp.float32),
            pltpu.SemaphoreType.DMA,
            pltpu.SemaphoreType.DMA,
        ],
        compiler_params=pltpu.CompilerParams(
            use_tc_tiling_on_sc=False, needs_layout_passes=False),
    )
    def gather_kernel(xv_hbm, tab_hbm, out_hbm, idx_v, rid_v, gbuf, tbuf,
                      gsem, ssem):
        wid = lax.axis_index("s") * NC + lax.axis_index("c")
        pltpu.sync_copy(xv_hbm.at[:, wid], idx_v)
        lanes = lax.iota(jnp.int32, _L)

        def fill_rids_and_gather(j, b):
            tr = lax.shift_right_logical(j, 3)
            j8 = j & 7
            for t0 in range(0, _CH, _L):
                rid_v[b, pl.ds(t0, _L)] = idx_v[tr, j8, pl.ds(t0, _L)]
            pltpu.async_copy(tab_hbm.at[rid_v.at[b]], gbuf.at[b], gsem)

        for b in range(_NB):
            fill_rids_and_gather(jnp.int32(b), b)

        zero = lanes * 0

        def group(g, carry):
            j0 = g * _NB
            for b in range(_NB):
                j = j0 + b
                gb = gbuf.at[b]
                tb = tbuf.at[b]
                pltpu.make_async_copy(
                    tab_hbm.at[pl.ds(0, _CH)], gb, gsem).wait()

                @pl.when(j >= _NB)
                def _():
                    pltpu.make_async_copy(
                        tb, out_hbm.at[0, :, 0], ssem).wait()

                for t0 in range(0, _CH, _L):
                    rows = lanes + t0
                    for c in range(D):
                        v = plsc.load_gather(gb, [rows, zero + c])
                        tb[c // 8, c % 8, pl.ds(t0, _L)] = v

                pltpu.async_copy(tb, out_hbm.at[j, :, wid], ssem)

                @pl.when(j + _NB < NJ)
                def _():
                    fill_rids_and_gather(j + _NB, b)

            return carry

        lax.fori_loop(0, NJ // _NB, group, 0)
        for b in range(_NB):
            pltpu.make_async_copy(
                tbuf.at[b], out_hbm.at[0, :, 0], ssem).wait()

    return gather_kernel


def kernel(x, table):
    S0, S1 = x.shape
    V, D = table.shape
    xv = (x.astype(jnp.int32)
          .reshape(S0 // _CH, _CH, S1 // 8, 8)
          .transpose(2, 0, 3, 1))
    outp = _build(S0, S1, V, D)(xv, table)
    return outp.transpose(2, 4, 0, 1, 3).reshape(S0, S1, D)


# speedup vs baseline: 1.7243x; 1.7243x over previous
import functools

import jax
import jax.numpy as jnp
from jax import lax
from jax.experimental import pallas as pl
from jax.experimental.pallas import tpu as pltpu
from jax.experimental.pallas import tpu_sc as plsc

_L = 16
_CH = 128
_NB = 4


@functools.cache
def _build(NI, NJ, V, D):
    info = plsc.get_sparse_core_info()
    NC, NS = info.num_cores, info.num_subcores
    NW = NC * NS
    NP = _CH // 2
    NT = 2 * NJ
    assert NI == NW * _CH and NJ == 200 and D == 64
    mesh = plsc.VectorSubcoreMesh(core_axis_name="c", subcore_axis_name="s")

    @functools.partial(
        pl.kernel,
        out_type=jax.ShapeDtypeStruct((NI, NJ, D), jnp.float32),
        mesh=mesh,
        scratch_types=[
            pltpu.VMEM((NJ // 8, 8, _CH), jnp.int32),
            pltpu.VMEM((_NB, NT), jnp.int32),
            pltpu.VMEM((_NB, 2, NJ, D), jnp.float32),
            pltpu.SemaphoreType.DMA,
            pltpu.SemaphoreType.DMA,
        ],
        compiler_params=pltpu.CompilerParams(
            use_tc_tiling_on_sc=False, needs_layout_passes=False),
    )
    def gather_kernel(xv_hbm, tab_hbm, out_hbm, idx_v, rid_v, gbuf,
                      gsem, ssem):
        wid = lax.axis_index("s") * NC + lax.axis_index("c")
        i0 = wid * _CH
        pltpu.sync_copy(xv_hbm.at[:, wid], idx_v)
        lanes = lax.iota(jnp.int32, _L)

        def fill_and_gather(k, b):
            l0 = 2 * k
            for g in range(NT // _L):
                pv = lanes + (_L * g)
                if _L * (g + 1) <= NJ:
                    jv, lv = pv, lanes * 0 + l0
                elif _L * g >= NJ:
                    jv, lv = pv - NJ, lanes * 0 + (l0 + 1)
                else:
                    oi = jnp.where(pv >= NJ, 1, 0).astype(jnp.int32)
                    jv, lv = pv - NJ * oi, oi + l0
                tr = lax.shift_right_logical(jv, 3)
                v = plsc.load_gather(idx_v, [tr, jv & 7, lv])
                rid_v[b, pl.ds(_L * g, _L)] = v
            for h in range(2):
                for off, n in ((0, 128), (128, NJ - 128)):
                    pltpu.async_copy(
                        tab_hbm.at[rid_v.at[b, pl.ds(NJ * h + off, n)]],
                        gbuf.at[b, h, pl.ds(off, n)], gsem)

        for b in range(2):
            fill_and_gather(jnp.int32(b), b)

        def step(k, carry):
            for b in range(_NB):
                kk = _NB * k + b
                for _h in range(2):
                    for off, n in ((0, 128), (128, NJ - 128)):
                        pltpu.make_async_copy(
                            tab_hbm.at[pl.ds(0, n)],
                            gbuf.at[0, 0, pl.ds(off, n)], gsem).wait()
                for h in range(2):
                    pltpu.async_copy(
                        gbuf.at[b, h], out_hbm.at[i0 + 2 * kk + h], ssem)

                @pl.when(kk >= 2)
                def _():
                    for _h in range(2):
                        pltpu.make_async_copy(
                            gbuf.at[0, 0], out_hbm.at[0], ssem).wait()

                @pl.when(kk + 2 < NP)
                def _():
                    fill_and_gather(kk + 2, (kk + 2) % _NB)

            return carry

        assert NP % _NB == 0
        lax.fori_loop(0, NP // _NB, step, 0)
        for _ in range(4):
            pltpu.make_async_copy(
                gbuf.at[0, 0], out_hbm.at[0], ssem).wait()

    return gather_kernel


def kernel(x, table):
    S0, S1 = x.shape
    V, D = table.shape
    xv = (x.astype(jnp.int32)
          .reshape(S0 // _CH, _CH, S1 // 8, 8)
          .transpose(2, 0, 3, 1))
    return _build(S0, S1, V, D)(xv, table)
